# baseline (speedup 1.0000x reference)
import functools

import jax
import jax.numpy as jnp
from jax import lax
from jax.experimental import pallas as pl
from jax.experimental.pallas import tpu as pltpu
from jax.experimental.pallas import tpu_sc as plsc

N_NODES = 10000
N_EDGES = 320000
IN_CH = 128
OUT_CH = 32
HEADS = 4

NC = 2
NS = 16
NW = NC * NS
E_PER_W = N_EDGES // NW
SUB = 2000
N_SUB = E_PER_W // SUB
LANES = 16

ROW_BLOCK = 2000


def _alpha_body(x_ref, w_ref, b_ref, attl_ref, attr_ref, al_ref, ar_ref):
    x = x_ref[...]
    h = lax.dot_general(x, w_ref[...], (((1,), (1,)), ((), ())),
                        preferred_element_type=jnp.float32,
                        precision=lax.Precision.HIGHEST) + b_ref[...]
    row = lax.broadcasted_iota(jnp.int32, (IN_CH, HEADS), 0)
    col = lax.broadcasted_iota(jnp.int32, (IN_CH, HEADS), 1)
    seg = (row >= col * OUT_CH) & (row < (col + 1) * OUT_CH)
    sl = jnp.where(seg, jnp.broadcast_to(attl_ref[...], (IN_CH, HEADS)), 0.0)
    sr = jnp.where(seg, jnp.broadcast_to(attr_ref[...], (IN_CH, HEADS)), 0.0)
    al_ref[...] = lax.dot_general(h, sl, (((1,), (0,)), ((), ())),
                                  preferred_element_type=jnp.float32,
                                  precision=lax.Precision.HIGHEST)
    ar_ref[...] = lax.dot_general(h, sr, (((1,), (0,)), ((), ())),
                                  preferred_element_type=jnp.float32,
                                  precision=lax.Precision.HIGHEST)


_alpha_call = pl.pallas_call(
    _alpha_body,
    grid=(N_NODES // ROW_BLOCK,),
    in_specs=[
        pl.BlockSpec((ROW_BLOCK, IN_CH), lambda i: (i, 0)),
        pl.BlockSpec((IN_CH, IN_CH), lambda i: (0, 0)),
        pl.BlockSpec((1, IN_CH), lambda i: (0, 0)),
        pl.BlockSpec((IN_CH, 1), lambda i: (0, 0)),
        pl.BlockSpec((IN_CH, 1), lambda i: (0, 0)),
    ],
    out_specs=[
        pl.BlockSpec((ROW_BLOCK, HEADS), lambda i: (i, 0)),
        pl.BlockSpec((ROW_BLOCK, HEADS), lambda i: (i, 0)),
    ],
    out_shape=[
        jax.ShapeDtypeStruct((N_NODES, HEADS), jnp.float32),
        jax.ShapeDtypeStruct((N_NODES, HEADS), jnp.float32),
    ],
)


def _edge_body(al_hbm, ar_hbm, src_hbm, dst_hbm, out_hbm,
               al_v, ar_v, src_v, dst_v, out_v):
    wid = lax.axis_index("s") * NC + lax.axis_index("c")
    pltpu.sync_copy(al_hbm, al_v)
    pltpu.sync_copy(ar_hbm, ar_v)
    lane = lax.iota(jnp.int32, LANES)

    for s_idx in range(N_SUB):
        base = wid * E_PER_W + s_idx * SUB
        pltpu.sync_copy(src_hbm.at[pl.ds(base, SUB)], src_v)
        pltpu.sync_copy(dst_hbm.at[pl.ds(base, SUB)], dst_v)

        def body(j, carry):
            sv = src_v[pl.ds(j * LANES, LANES)] * HEADS
            dv = dst_v[pl.ds(j * LANES, LANES)] * HEADS
            obase = j * (LANES * HEADS)
            oidx = obase + lane * HEADS
            for hd in range(HEADS):
                a = plsc.load_gather(al_v, [sv + hd])
                r = plsc.load_gather(ar_v, [dv + hd])
                v = a + r
                res = jnp.where(v >= 0.0, v, v * jnp.float32(0.01))
                plsc.store_scatter(out_v, [oidx + hd], res)
            return carry

        lax.fori_loop(0, SUB // LANES, body, 0)
        pltpu.sync_copy(out_v, out_hbm.at[pl.ds(base * HEADS, SUB * HEADS)])


@functools.cache
def _edge_kernel():
    return pl.kernel(
        _edge_body,
        mesh=plsc.VectorSubcoreMesh(core_axis_name="c", subcore_axis_name="s",
                                    num_cores=NC, num_subcores=NS),
        compiler_params=pltpu.CompilerParams(needs_layout_passes=False),
        out_type=jax.ShapeDtypeStruct((N_EDGES * HEADS,), jnp.float32),
        scratch_types=[
            pltpu.VMEM((N_NODES * HEADS,), jnp.float32),
            pltpu.VMEM((N_NODES * HEADS,), jnp.float32),
            pltpu.VMEM((SUB,), jnp.int32),
            pltpu.VMEM((SUB,), jnp.int32),
            pltpu.VMEM((SUB * HEADS,), jnp.float32),
        ],
    )


def kernel(x, edge_index, W, b, att_l, att_r):
    src = edge_index[0].astype(jnp.int32)
    dst = edge_index[1].astype(jnp.int32)
    alpha_l, alpha_r = _alpha_call(
        x, W, b.reshape(1, IN_CH),
        att_l.reshape(IN_CH, 1), att_r.reshape(IN_CH, 1))
    out_flat = _edge_kernel()(alpha_l.reshape(-1), alpha_r.reshape(-1), src, dst)
    return out_flat.reshape(N_EDGES, HEADS)


# speedup vs baseline: 3.7472x; 3.7472x over previous
import functools

import jax
import jax.numpy as jnp
from jax import lax
from jax.experimental import pallas as pl
from jax.experimental.pallas import tpu as pltpu
from jax.experimental.pallas import tpu_sc as plsc

N_NODES = 10000
N_EDGES = 320000
IN_CH = 128
OUT_CH = 32
HEADS = 4
H2 = 2 * HEADS

NC = 2
NS = 16
NW = NC * NS
LANES = 16

C_EDGES = 2560
N_CHUNKS = N_EDGES // C_EDGES
CHUNKS_PER_W = -(-N_CHUNKS // NW)

ROW_BLOCK = 2000


def _alpha_body(x_ref, w_ref, b_ref, attl_ref, attr_ref, out_ref):
    row = lax.broadcasted_iota(jnp.int32, (IN_CH, H2), 0)
    col = lax.broadcasted_iota(jnp.int32, (IN_CH, H2), 1)
    seg_l = (col < HEADS) & (row >= col * OUT_CH) & (row < (col + 1) * OUT_CH)
    cr = col - HEADS
    seg_r = (col >= HEADS) & (row >= cr * OUT_CH) & (row < (cr + 1) * OUT_CH)
    s = (jnp.where(seg_l, jnp.broadcast_to(attl_ref[...], (IN_CH, H2)), 0.0)
         + jnp.where(seg_r, jnp.broadcast_to(attr_ref[...], (IN_CH, H2)), 0.0))
    a = lax.dot_general(w_ref[...], s, (((0,), (0,)), ((), ())),
                        preferred_element_type=jnp.float32,
                        precision=lax.Precision.HIGHEST)
    c = lax.dot_general(b_ref[...], s, (((1,), (0,)), ((), ())),
                        preferred_element_type=jnp.float32,
                        precision=lax.Precision.HIGHEST)
    out_ref[...] = lax.dot_general(x_ref[...], a, (((1,), (0,)), ((), ())),
                                   preferred_element_type=jnp.float32,
                                   precision=lax.Precision.HIGHEST) + c


_alpha_call = pl.pallas_call(
    _alpha_body,
    grid=(N_NODES // ROW_BLOCK,),
    in_specs=[
        pl.BlockSpec((ROW_BLOCK, IN_CH), lambda i: (i, 0)),
        pl.BlockSpec((IN_CH, IN_CH), lambda i: (0, 0)),
        pl.BlockSpec((1, IN_CH), lambda i: (0, 0)),
        pl.BlockSpec((IN_CH, 1), lambda i: (0, 0)),
        pl.BlockSpec((IN_CH, 1), lambda i: (0, 0)),
    ],
    out_specs=pl.BlockSpec((ROW_BLOCK, H2), lambda i: (i, 0)),
    out_shape=jax.ShapeDtypeStruct((N_NODES, H2), jnp.float32),
)


def _edge_body(tab_hbm, src_hbm, dst_hbm, out_hbm,
               tab_v, src_v, dst_v, out_v):
    wid = lax.axis_index("s") * NC + lax.axis_index("c")
    pltpu.sync_copy(tab_hbm, tab_v)
    lane = lax.iota(jnp.int32, LANES)

    def body(j, carry):
        sv = src_v[pl.ds(j * LANES, LANES)] * H2
        dv = dst_v[pl.ds(j * LANES, LANES)] * H2
        obase = (j // 8) * (HEADS * 128) + (j % 8) * LANES + lane
        for hd in range(HEADS):
            a = plsc.load_gather(tab_v, [sv + hd])
            r = plsc.load_gather(tab_v, [dv + (HEADS + hd)])
            v = a + r
            res = jnp.where(v >= 0.0, v, v * jnp.float32(0.01))
            plsc.store_scatter(out_v, [obase + hd * 128], res)
        return carry

    for t in range(CHUNKS_PER_W):
        cid = t * NW + wid

        @pl.when(cid < N_CHUNKS)
        def _():
            base_e = cid * C_EDGES
            pltpu.sync_copy(src_hbm.at[pl.ds(base_e, C_EDGES)], src_v)
            pltpu.sync_copy(dst_hbm.at[pl.ds(base_e, C_EDGES)], dst_v)
            lax.fori_loop(0, C_EDGES // LANES, body, 0)
            pltpu.sync_copy(
                out_v, out_hbm.at[pl.ds(base_e * HEADS, C_EDGES * HEADS)])


@functools.cache
def _edge_kernel():
    return pl.kernel(
        _edge_body,
        mesh=plsc.VectorSubcoreMesh(core_axis_name="c", subcore_axis_name="s",
                                    num_cores=NC, num_subcores=NS),
        compiler_params=pltpu.CompilerParams(needs_layout_passes=False),
        out_type=jax.ShapeDtypeStruct((N_EDGES * HEADS,), jnp.float32),
        scratch_types=[
            pltpu.VMEM((N_NODES * H2,), jnp.float32),
            pltpu.VMEM((C_EDGES,), jnp.int32),
            pltpu.VMEM((C_EDGES,), jnp.int32),
            pltpu.VMEM((C_EDGES * HEADS,), jnp.float32),
        ],
    )


def kernel(x, edge_index, W, b, att_l, att_r):
    src = edge_index[0].astype(jnp.int32)
    dst = edge_index[1].astype(jnp.int32)
    alpha = _alpha_call(x, W, b.reshape(1, IN_CH),
                        att_l.reshape(IN_CH, 1), att_r.reshape(IN_CH, 1))
    out_flat = _edge_kernel()(alpha.reshape(-1), src, dst)
    return (out_flat.reshape(N_EDGES // 128, HEADS, 128)
            .transpose(0, 2, 1).reshape(N_EDGES, HEADS))


# speedup vs baseline: 5.2178x; 1.3924x over previous
import functools

import jax
import jax.numpy as jnp
from jax import lax
from jax.experimental import pallas as pl
from jax.experimental.pallas import tpu as pltpu
from jax.experimental.pallas import tpu_sc as plsc

N_NODES = 10000
N_EDGES = 320000
IN_CH = 128
OUT_CH = 32
HEADS = 4
H2 = 2 * HEADS

NC = 2
NS = 16
NW = NC * NS
LANES = 16

C_EDGES = 2560
N_CHUNKS = N_EDGES // C_EDGES
CHUNKS_PER_W = -(-N_CHUNKS // NW)

ROW_BLOCK = 2000


def _alpha_body(x_ref, w_ref, b_ref, attl_ref, attr_ref, out_ref):
    row = lax.broadcasted_iota(jnp.int32, (IN_CH, H2), 0)
    col = lax.broadcasted_iota(jnp.int32, (IN_CH, H2), 1)
    seg_l = (col < HEADS) & (row >= col * OUT_CH) & (row < (col + 1) * OUT_CH)
    cr = col - HEADS
    seg_r = (col >= HEADS) & (row >= cr * OUT_CH) & (row < (cr + 1) * OUT_CH)
    s = (jnp.where(seg_l, jnp.broadcast_to(attl_ref[...], (IN_CH, H2)), 0.0)
         + jnp.where(seg_r, jnp.broadcast_to(attr_ref[...], (IN_CH, H2)), 0.0))
    a = lax.dot_general(w_ref[...], s, (((0,), (0,)), ((), ())),
                        preferred_element_type=jnp.float32,
                        precision=lax.Precision.HIGHEST)
    c = lax.dot_general(b_ref[...], s, (((1,), (0,)), ((), ())),
                        preferred_element_type=jnp.float32,
                        precision=lax.Precision.HIGHEST)
    out_ref[...] = lax.dot_general(x_ref[...], a, (((1,), (0,)), ((), ())),
                                   preferred_element_type=jnp.float32,
                                   precision=lax.Precision.HIGHEST) + c


_alpha_call = pl.pallas_call(
    _alpha_body,
    grid=(N_NODES // ROW_BLOCK,),
    in_specs=[
        pl.BlockSpec((ROW_BLOCK, IN_CH), lambda i: (i, 0)),
        pl.BlockSpec((IN_CH, IN_CH), lambda i: (0, 0)),
        pl.BlockSpec((1, IN_CH), lambda i: (0, 0)),
        pl.BlockSpec((IN_CH, 1), lambda i: (0, 0)),
        pl.BlockSpec((IN_CH, 1), lambda i: (0, 0)),
    ],
    out_specs=pl.BlockSpec((ROW_BLOCK, H2), lambda i: (i, 0)),
    out_shape=jax.ShapeDtypeStruct((N_NODES, H2), jnp.float32),
)


def _edge_body(tab_hbm, ei_hbm, out_hbm, tab_v, ei_v, out_v):
    wid = lax.axis_index("s") * NC + lax.axis_index("c")
    pltpu.sync_copy(tab_hbm, tab_v)
    lane = lax.iota(jnp.int32, LANES)

    for t in range(CHUNKS_PER_W):
        cid = t * NW + wid

        @pl.when(cid < N_CHUNKS)
        def _():
            base_e = cid * C_EDGES
            pltpu.sync_copy(ei_hbm.at[pl.ds(base_e * 2, C_EDGES * 2)], ei_v)

            @plsc.parallel_loop(0, C_EDGES // LANES, unroll=2)
            def body(j):
                boff = (j // 8) * 256 + (j % 8) * LANES
                sv = ei_v[pl.ds(boff, LANES)] * H2
                dv = ei_v[pl.ds(boff + 128, LANES)] * H2
                obase = (j // 8) * (HEADS * 128) + (j % 8) * LANES + lane
                for hd in range(HEADS):
                    a = plsc.load_gather(tab_v, [sv + hd])
                    r = plsc.load_gather(tab_v, [dv + (HEADS + hd)])
                    v = a + r
                    res = jnp.where(v >= 0.0, v, v * jnp.float32(0.01))
                    plsc.store_scatter(out_v, [obase + hd * 128], res)

            pltpu.sync_copy(
                out_v, out_hbm.at[pl.ds(base_e * HEADS, C_EDGES * HEADS)])


@functools.cache
def _edge_kernel():
    return pl.kernel(
        _edge_body,
        mesh=plsc.VectorSubcoreMesh(core_axis_name="c", subcore_axis_name="s",
                                    num_cores=NC, num_subcores=NS),
        compiler_params=pltpu.CompilerParams(needs_layout_passes=False),
        out_type=jax.ShapeDtypeStruct((N_EDGES * HEADS,), jnp.float32),
        scratch_types=[
            pltpu.VMEM((N_NODES * H2,), jnp.float32),
            pltpu.VMEM((C_EDGES * 2,), jnp.int32),
            pltpu.VMEM((C_EDGES * HEADS,), jnp.float32),
        ],
    )


def kernel(x, edge_index, W, b, att_l, att_r):
    eif = (edge_index.astype(jnp.int32)
           .reshape(2, N_EDGES // 128, 128)
           .transpose(1, 0, 2).reshape(-1))
    alpha = _alpha_call(x, W, b.reshape(1, IN_CH),
                        att_l.reshape(IN_CH, 1), att_r.reshape(IN_CH, 1))
    out_flat = _edge_kernel()(alpha.reshape(-1), eif)
    return (out_flat.reshape(N_EDGES // 128, HEADS, 128)
            .transpose(0, 2, 1).reshape(N_EDGES, HEADS))
